# baseline (speedup 1.0000x reference)
import jax
import jax.numpy as jnp
from jax.experimental import pallas as pl
from jax.experimental.pallas import tpu as pltpu

N_NODES = 10000
D = 128
N_REL = 16
N_EDGES = 320000
H1, H2, DOUT = 256, 128, 64
ROW_BLK = 400


def _mlp_body(a_ref, w1_ref, b1_ref, w2_ref, b2_ref, w3_ref, b3_ref, o_ref):
    x = jnp.tanh(
        jnp.dot(a_ref[...], w1_ref[...], preferred_element_type=jnp.float32)
        + b1_ref[...]
    )
    x = jnp.tanh(
        jnp.dot(x, w2_ref[...], preferred_element_type=jnp.float32) + b2_ref[...]
    )
    o_ref[...] = (
        jnp.dot(x, w3_ref[...], preferred_element_type=jnp.float32) + b3_ref[...]
    )


def _mlp(acc, W1, b1, W2, b2, W3, b3):
    n = acc.shape[0]
    grid = n // ROW_BLK
    return pl.pallas_call(
        _mlp_body,
        grid=(grid,),
        in_specs=[
            pl.BlockSpec((ROW_BLK, D), lambda i: (i, 0)),
            pl.BlockSpec((D, H1), lambda i: (0, 0)),
            pl.BlockSpec((1, H1), lambda i: (0, 0)),
            pl.BlockSpec((H1, H2), lambda i: (0, 0)),
            pl.BlockSpec((1, H2), lambda i: (0, 0)),
            pl.BlockSpec((H2, DOUT), lambda i: (0, 0)),
            pl.BlockSpec((1, DOUT), lambda i: (0, 0)),
        ],
        out_specs=pl.BlockSpec((ROW_BLK, DOUT), lambda i: (i, 0)),
        out_shape=jax.ShapeDtypeStruct((n, DOUT), jnp.float32),
    )(acc, W1, b1[None, :], W2, b2[None, :], W3, b3[None, :])


def _l2norm(x):
    return x / (jnp.linalg.norm(x, axis=-1, keepdims=True) + 1e-9)


def _transfer(e, e_t, r_t):
    return _l2norm(e + jnp.sum(e * e_t, axis=-1, keepdims=True) * r_t)


def kernel(ent_emb, ent_transfer, rel_emb, rel_transfer, edge_index, edge_rel,
           edge_dir, W1, b1, W2, b2, W3, b3):
    head = edge_index[0]
    tail = edge_index[1]
    N = ent_emb.shape[0]
    r_t = rel_transfer[edge_rel]
    r_e = _l2norm(rel_emb[edge_rel])
    h_p = _transfer(ent_emb[head], ent_transfer[head], r_t)
    t_p = _transfer(ent_emb[tail], ent_transfer[tail], r_t)
    dirn = jnp.sign(edge_dir - 0.5)
    t_msg = jnp.tanh(t_p - dirn[:, None] * r_e)
    score = jnp.sum(h_p * t_msg, axis=-1)
    seg_max = jax.ops.segment_max(score, head, num_segments=N)
    seg_max = jnp.where(jnp.isfinite(seg_max), seg_max, 0.0)
    ex = jnp.exp(score - seg_max[head])
    denom = jax.ops.segment_sum(ex, head, num_segments=N)
    w = ex / (denom[head] + 1e-9)
    h = ent_emb
    acc = h
    for _ in range(2):
        msg = w[:, None] * h[tail]
        h = jax.ops.segment_sum(msg, head, num_segments=N)
        acc = acc + h
    return _mlp(acc, W1, b1, W2, b2, W3, b3)


# speedup vs baseline: 4.5814x; 4.5814x over previous
import jax
import jax.numpy as jnp
from jax import lax
from jax.experimental import pallas as pl
from jax.experimental.pallas import tpu as pltpu
from jax.experimental.pallas import tpu_sc as plsc

N = 10000
D = 128
R = 16
E = 320000
H1, H2, DOUT = 256, 128, 64

_NC, _NS = 2, 16
_NW = _NC * _NS
_E2 = E // 2
_EPTG = _E2 // _NW
_EPTS = _E2 // _NS
_BG = 200
_B = 400
_BD = 1000
_NP = 10240
_NQ = 2560
_NA = _NQ + 128
_RZ = _NA // _NS
_RPT = _NQ // _NS
_EB = 1600
_ROW_BLK = 400

f32 = jnp.float32
i32 = jnp.int32


def _mesh():
    return plsc.VectorSubcoreMesh(
        core_axis_name="c", subcore_axis_name="s",
        num_cores=_NC, num_subcores=_NS)


def _zero_vmem(ref, nrows, ncolv):
    z = jnp.zeros((16,), f32)

    def body(r, _):
        for j in range(ncolv):
            ref[r, pl.ds(j * 16, 16)] = z
        return 0

    lax.fori_loop(0, nrows, body, 0)


def _paired_blocks(nblk, fn):
    lax.fori_loop(0, nblk // 2, lambda g, c: fn(g, 2, c), 0)
    if nblk % 2:
        fn(nblk // 2, 1, 0)


def _gather_body(off, emb, head, tail, gh_out, gt_out,
                 ih0, it0, ih1, it1, rh0, rt0, rh1, rt1,
                 s0, s1, s2, s3, w0, w1, w2, w3):
    cid = lax.axis_index("c")
    sid = lax.axis_index("s")
    ebase = (cid * _NS + sid) * _EPTG
    ihs, its = (ih0, ih1), (it0, it1)
    rhs, rts = (rh0, rh1), (rt0, rt1)
    rsem = (s0, s1, s2, s3)
    wsem = (w0, w1, w2, w3)

    def pair(g, nb, _):
        rds = []
        for k in range(nb):
            base = ebase + (g * 2 + k) * _BG
            pltpu.sync_copy(head.at[pl.ds(off + base, _BG)], ihs[k])
            pltpu.sync_copy(tail.at[pl.ds(off + base, _BG)], its[k])
            rds.append(pltpu.async_copy(emb.at[ihs[k]], rhs[k], rsem[2 * k]))
            rds.append(pltpu.async_copy(emb.at[its[k]], rts[k],
                                        rsem[2 * k + 1]))
        wrs = []
        for k in range(nb):
            base = ebase + (g * 2 + k) * _BG
            rds[2 * k].wait()
            wrs.append(pltpu.async_copy(rhs[k], gh_out.at[pl.ds(base, _BG)],
                                        wsem[2 * k]))
            rds[2 * k + 1].wait()
            wrs.append(pltpu.async_copy(rts[k], gt_out.at[pl.ds(base, _BG)],
                                        wsem[2 * k + 1]))
        for w in wrs:
            w.wait()
        return 0

    _paired_blocks(_EPTG // _BG, pair)


def _gather(emb, head, tail, off):
    kern = pl.kernel(
        lambda *a: _gather_body(off, *a),
        out_type=(
            jax.ShapeDtypeStruct((_E2, D), f32),
            jax.ShapeDtypeStruct((_E2, D), f32),
        ),
        mesh=_mesh(),
        scratch_types=(
            pltpu.VMEM((_BG,), i32), pltpu.VMEM((_BG,), i32),
            pltpu.VMEM((_BG,), i32), pltpu.VMEM((_BG,), i32),
            pltpu.VMEM((_BG, D), f32), pltpu.VMEM((_BG, D), f32),
            pltpu.VMEM((_BG, D), f32), pltpu.VMEM((_BG, D), f32),
            pltpu.SemaphoreType.DMA, pltpu.SemaphoreType.DMA,
            pltpu.SemaphoreType.DMA, pltpu.SemaphoreType.DMA,
            pltpu.SemaphoreType.DMA, pltpu.SemaphoreType.DMA,
            pltpu.SemaphoreType.DMA, pltpu.SemaphoreType.DMA,
        ),
    )
    return kern(emb, head, tail)


def _gather_s_body(off, s16, head, tail, sh_out, st_out,
                   ih0, it0, ih1, it1, sh0, st0, sh1, st1,
                   s0, s1, s2, s3, w0, w1, w2, w3):
    cid = lax.axis_index("c")
    sid = lax.axis_index("s")
    ebase = (cid * _NS + sid) * _EPTG
    ihs, its = (ih0, ih1), (it0, it1)
    shs, sts = (sh0, sh1), (st0, st1)
    rsem = (s0, s1, s2, s3)
    wsem = (w0, w1, w2, w3)

    def pair(g, nb, _):
        rds = []
        for k in range(nb):
            base = ebase + (g * 2 + k) * _BG
            pltpu.sync_copy(head.at[pl.ds(off + base, _BG)], ihs[k])
            pltpu.sync_copy(tail.at[pl.ds(off + base, _BG)], its[k])
            rds.append(pltpu.async_copy(s16.at[ihs[k]], shs[k], rsem[2 * k]))
            rds.append(pltpu.async_copy(s16.at[its[k]], sts[k],
                                        rsem[2 * k + 1]))
        wrs = []
        for k in range(nb):
            base = ebase + (g * 2 + k) * _BG
            rds[2 * k].wait()
            wrs.append(pltpu.async_copy(shs[k], sh_out.at[pl.ds(base, _BG)],
                                        wsem[2 * k]))
            rds[2 * k + 1].wait()
            wrs.append(pltpu.async_copy(sts[k], st_out.at[pl.ds(base, _BG)],
                                        wsem[2 * k + 1]))
        for w in wrs:
            w.wait()
        return 0

    _paired_blocks(_EPTG // _BG, pair)


def _gather_s(s16, head, tail, off):
    kern = pl.kernel(
        lambda *a: _gather_s_body(off, *a),
        out_type=(
            jax.ShapeDtypeStruct((_E2, 16), f32),
            jax.ShapeDtypeStruct((_E2, 16), f32),
        ),
        mesh=_mesh(),
        compiler_params=pltpu.CompilerParams(use_tc_tiling_on_sc=False),
        scratch_types=(
            pltpu.VMEM((_BG,), i32), pltpu.VMEM((_BG,), i32),
            pltpu.VMEM((_BG,), i32), pltpu.VMEM((_BG,), i32),
            pltpu.VMEM((_BG, 16), f32), pltpu.VMEM((_BG, 16), f32),
            pltpu.VMEM((_BG, 16), f32), pltpu.VMEM((_BG, 16), f32),
            pltpu.SemaphoreType.DMA, pltpu.SemaphoreType.DMA,
            pltpu.SemaphoreType.DMA, pltpu.SemaphoreType.DMA,
            pltpu.SemaphoreType.DMA, pltpu.SemaphoreType.DMA,
            pltpu.SemaphoreType.DMA, pltpu.SemaphoreType.DMA,
        ),
    )
    return kern(s16, head, tail)


def _clamp_idx(idx_ref, q, nv, dump):
    def body(t, _):
        v = idx_ref[pl.ds(t * 16, 16)] - q * _NQ
        ok = jnp.logical_and(v >= 0, v < _NQ)
        idx_ref[pl.ds(t * 16, 16)] = jnp.where(ok, v, dump)
        return 0

    lax.fori_loop(0, nv, body, 0)


def _scatter_rows_body(off, m1, head, u_out,
                       ih0, ih1, pay0, pay1, u_s,
                       s0, s1, w0, w1):
    cid = lax.axis_index("c")
    sid = lax.axis_index("s")
    ebase = sid * _EPTS
    ihs = (ih0, ih1)
    pays = (pay0, pay1)
    rsem = (s0, s1)
    wsem = (w0, w1)

    _zero_vmem(pay0, _RZ, D // 16)
    for p in range(2):
        q = cid * 2 + p
        pltpu.sync_copy(pay0.at[pl.ds(0, _RZ)],
                        u_s.at[pl.ds(sid * _RZ, _RZ)])
        plsc.subcore_barrier()

        def pair(g, nb, _):
            rds = []
            for k in range(nb):
                base = ebase + (g * 2 + k) * _B
                pltpu.sync_copy(head.at[pl.ds(off + base, _B)], ihs[k])
                _clamp_idx(ihs[k], q, _B // 16, _NQ + sid * 2 + k)
                rds.append(pltpu.async_copy(m1.at[pl.ds(base, _B)], pays[k],
                                            rsem[k]))
            wrs = []
            for k in range(nb):
                rds[k].wait()
                wrs.append(pltpu.async_copy(pays[k], u_s.at[ihs[k]], wsem[k],
                                            add=True))
            for w in wrs:
                w.wait()
            return 0

        _paired_blocks(_EPTS // _B, pair)
        plsc.subcore_barrier()
        pltpu.sync_copy(u_s.at[pl.ds(sid * _RPT, _RPT)],
                        u_out.at[q, pl.ds(sid * _RPT, _RPT)])
        plsc.subcore_barrier()
        if p == 0:
            _zero_vmem(pay0, _RZ, D // 16)


def _scatter_rows(m1, head, off):
    kern = pl.kernel(
        lambda *a: _scatter_rows_body(off, *a),
        out_type=jax.ShapeDtypeStruct((4, _NQ, D), f32),
        mesh=_mesh(),
        scratch_types=(
            pltpu.VMEM((_B,), i32), pltpu.VMEM((_B,), i32),
            pltpu.VMEM((_B, D), f32), pltpu.VMEM((_B, D), f32),
            pltpu.VMEM_SHARED((_NA, D), f32),
            pltpu.SemaphoreType.DMA, pltpu.SemaphoreType.DMA,
            pltpu.SemaphoreType.DMA, pltpu.SemaphoreType.DMA,
        ),
    )
    return kern(m1, head)


def _scatter_den_body(off, ex16, head, den_out,
                      ih0, ih1, ex0, ex1, den_s,
                      s0, s1, w0, w1):
    cid = lax.axis_index("c")
    sid = lax.axis_index("s")
    ebase = (cid * _NS + sid) * _EPTG
    ihs = (ih0, ih1)
    exs = (ex0, ex1)
    rsem = (s0, s1)
    wsem = (w0, w1)

    _zero_vmem(ex0, _NP // _NS, 1)
    pltpu.sync_copy(ex0.at[pl.ds(0, _NP // _NS)],
                    den_s.at[pl.ds(sid * (_NP // _NS), _NP // _NS)])
    plsc.subcore_barrier()

    def pair(g, nb, _):
        rds = []
        for k in range(nb):
            base = ebase + (g * 2 + k) * _BD
            pltpu.sync_copy(head.at[pl.ds(off + base, _BD)], ihs[k])
            rds.append(pltpu.async_copy(ex16.at[pl.ds(base, _BD)],
                                        exs[k], rsem[k]))
        wrs = []
        for k in range(nb):
            rds[k].wait()
            wrs.append(pltpu.async_copy(exs[k], den_s.at[ihs[k]], wsem[k],
                                        add=True))
        for w in wrs:
            w.wait()
        return 0

    _paired_blocks(_EPTG // _BD, pair)
    plsc.subcore_barrier()
    pltpu.sync_copy(den_s.at[pl.ds(sid * (_NP // _NS), _NP // _NS)],
                    den_out.at[cid, pl.ds(sid * (_NP // _NS), _NP // _NS)])


def _scatter_den(ex16, head, off):
    kern = pl.kernel(
        lambda *a: _scatter_den_body(off, *a),
        out_type=jax.ShapeDtypeStruct((_NC, _NP, 16), f32),
        mesh=_mesh(),
        compiler_params=pltpu.CompilerParams(use_tc_tiling_on_sc=False),
        scratch_types=(
            pltpu.VMEM((_BD,), i32), pltpu.VMEM((_BD,), i32),
            pltpu.VMEM((_BD, 16), f32), pltpu.VMEM((_BD, 16), f32),
            pltpu.VMEM_SHARED((_NP, 16), f32),
            pltpu.SemaphoreType.DMA, pltpu.SemaphoreType.DMA,
            pltpu.SemaphoreType.DMA, pltpu.SemaphoreType.DMA,
        ),
    )
    return kern(ex16, head)


def _gather2_body(off, h1, tail, g2_out,
                  it0, it1, r0, r1, s0, s1, w0, w1):
    cid = lax.axis_index("c")
    sid = lax.axis_index("s")
    ebase = (cid * _NS + sid) * _EPTG
    its = (it0, it1)
    rows = (r0, r1)
    rsem = (s0, s1)
    wsem = (w0, w1)

    def pair(g, nb, _):
        rds = []
        for k in range(nb):
            base = ebase + (g * 2 + k) * _BG
            pltpu.sync_copy(tail.at[pl.ds(off + base, _BG)], its[k])
            rds.append(pltpu.async_copy(h1.at[its[k]], rows[k], rsem[k]))
        wrs = []
        for k in range(nb):
            base = ebase + (g * 2 + k) * _BG
            rds[k].wait()
            wrs.append(pltpu.async_copy(rows[k], g2_out.at[pl.ds(base, _BG)],
                                        wsem[k]))
        for w in wrs:
            w.wait()
        return 0

    _paired_blocks(_EPTG // _BG, pair)


def _gather2(h1, tail, off):
    kern = pl.kernel(
        lambda *a: _gather2_body(off, *a),
        out_type=jax.ShapeDtypeStruct((_E2, D), f32),
        mesh=_mesh(),
        scratch_types=(
            pltpu.VMEM((_BG,), i32), pltpu.VMEM((_BG,), i32),
            pltpu.VMEM((_BG, D), f32), pltpu.VMEM((_BG, D), f32),
            pltpu.SemaphoreType.DMA, pltpu.SemaphoreType.DMA,
            pltpu.SemaphoreType.DMA, pltpu.SemaphoreType.DMA,
        ),
    )
    return kern(h1, tail)


def _prep_body(e_ref, et_ref, r_ref, s16_ref, ren_ref):
    s = jnp.sum(e_ref[...] * et_ref[...], axis=1, keepdims=True)
    s16_ref[...] = jnp.broadcast_to(s, (N, 16))
    rr = r_ref[...]
    nrm = jnp.sqrt(jnp.sum(rr * rr, axis=1, keepdims=True))
    ren_ref[...] = rr / (nrm + 1e-9)


def _prep(ent_emb, ent_transfer, rel_emb):
    return pl.pallas_call(
        _prep_body,
        out_shape=(
            jax.ShapeDtypeStruct((N, 16), f32),
            jax.ShapeDtypeStruct((R, D), f32),
        ),
    )(ent_emb, ent_transfer, rel_emb)


def _score_body(gh_ref, gt_ref, sh_ref, st_ref, rel_ref, dir_ref,
                rt_ref, ren_ref, m1_ref, ex_ref):
    onehot = (rel_ref[...] == lax.broadcasted_iota(i32, (1, R), 1)).astype(f32)
    rt = jnp.dot(onehot, rt_ref[...], preferred_element_type=f32)
    ren = jnp.dot(onehot, ren_ref[...], preferred_element_type=f32)
    s_h = sh_ref[...][:, 0:1]
    s_t = st_ref[...][:, 0:1]
    gt = gt_ref[...]
    hraw = gh_ref[...] + s_h * rt
    traw = gt + s_t * rt
    hp = hraw / (jnp.sqrt(jnp.sum(hraw * hraw, axis=1, keepdims=True)) + 1e-9)
    tp = traw / (jnp.sqrt(jnp.sum(traw * traw, axis=1, keepdims=True)) + 1e-9)
    dirn = jnp.sign(dir_ref[...] - 0.5)
    tmsg = jnp.tanh(tp - dirn * ren)
    score = jnp.sum(hp * tmsg, axis=1, keepdims=True)
    ex = jnp.exp(score)
    m1_ref[...] = ex * gt
    ex_ref[...] = jnp.broadcast_to(ex, (_EB, 16))


def _score(gh, gt, sh, st, rel2, dir2, rt, ren):
    grid = _E2 // _EB
    return pl.pallas_call(
        _score_body,
        grid=(grid,),
        in_specs=[
            pl.BlockSpec((_EB, D), lambda i: (i, 0)),
            pl.BlockSpec((_EB, D), lambda i: (i, 0)),
            pl.BlockSpec((_EB, 16), lambda i: (i, 0)),
            pl.BlockSpec((_EB, 16), lambda i: (i, 0)),
            pl.BlockSpec((_EB, 1), lambda i: (i, 0)),
            pl.BlockSpec((_EB, 1), lambda i: (i, 0)),
            pl.BlockSpec((R, D), lambda i: (0, 0)),
            pl.BlockSpec((R, D), lambda i: (0, 0)),
        ],
        out_specs=[
            pl.BlockSpec((_EB, D), lambda i: (i, 0)),
            pl.BlockSpec((_EB, 16), lambda i: (i, 0)),
        ],
        out_shape=[
            jax.ShapeDtypeStruct((_E2, D), f32),
            jax.ShapeDtypeStruct((_E2, 16), f32),
        ],
    )(gh, gt, sh, st, rel2, dir2, rt, ren)


def _mul_body(g2_ref, ex_ref, m2_ref):
    m2_ref[...] = g2_ref[...] * ex_ref[...][:, 0:1]


def _mul(g2, ex16):
    grid = _E2 // _EB
    return pl.pallas_call(
        _mul_body,
        grid=(grid,),
        in_specs=[
            pl.BlockSpec((_EB, D), lambda i: (i, 0)),
            pl.BlockSpec((_EB, 16), lambda i: (i, 0)),
        ],
        out_specs=pl.BlockSpec((_EB, D), lambda i: (i, 0)),
        out_shape=jax.ShapeDtypeStruct((_E2, D), f32),
    )(g2, ex16)


def _combine_body(a_ref, b_ref, den_ref, o_ref):
    o_ref[...] = (a_ref[...] + b_ref[...]) / (den_ref[...] + 1e-9)


def _combine(u1a, u1b, den):
    grid = N // _ROW_BLK
    return pl.pallas_call(
        _combine_body,
        grid=(grid,),
        in_specs=[
            pl.BlockSpec((_ROW_BLK, D), lambda i: (i, 0)),
            pl.BlockSpec((_ROW_BLK, D), lambda i: (i, 0)),
            pl.BlockSpec((_ROW_BLK, 1), lambda i: (i, 0)),
        ],
        out_specs=pl.BlockSpec((_ROW_BLK, D), lambda i: (i, 0)),
        out_shape=jax.ShapeDtypeStruct((N, D), f32),
    )(u1a, u1b, den)


def _final_body(e_ref, h1_ref, u2a_ref, u2b_ref, den_ref,
                w1_ref, b1_ref, w2_ref, b2_ref, w3_ref, b3_ref, o_ref):
    h2 = (u2a_ref[...] + u2b_ref[...]) / (den_ref[...] + 1e-9)
    acc = e_ref[...] + h1_ref[...] + h2
    x = jnp.tanh(jnp.dot(acc, w1_ref[...],
                         preferred_element_type=f32) + b1_ref[...])
    x = jnp.tanh(jnp.dot(x, w2_ref[...],
                         preferred_element_type=f32) + b2_ref[...])
    o_ref[...] = jnp.dot(x, w3_ref[...],
                         preferred_element_type=f32) + b3_ref[...]


def _final(ent_emb, h1, u2a, u2b, den, W1, b1, W2, b2, W3, b3):
    grid = N // _ROW_BLK
    return pl.pallas_call(
        _final_body,
        grid=(grid,),
        in_specs=[
            pl.BlockSpec((_ROW_BLK, D), lambda i: (i, 0)),
            pl.BlockSpec((_ROW_BLK, D), lambda i: (i, 0)),
            pl.BlockSpec((_ROW_BLK, D), lambda i: (i, 0)),
            pl.BlockSpec((_ROW_BLK, D), lambda i: (i, 0)),
            pl.BlockSpec((_ROW_BLK, 1), lambda i: (i, 0)),
            pl.BlockSpec((D, H1), lambda i: (0, 0)),
            pl.BlockSpec((1, H1), lambda i: (0, 0)),
            pl.BlockSpec((H1, H2), lambda i: (0, 0)),
            pl.BlockSpec((1, H2), lambda i: (0, 0)),
            pl.BlockSpec((H2, DOUT), lambda i: (0, 0)),
            pl.BlockSpec((1, DOUT), lambda i: (0, 0)),
        ],
        out_specs=pl.BlockSpec((_ROW_BLK, DOUT), lambda i: (i, 0)),
        out_shape=jax.ShapeDtypeStruct((N, DOUT), f32),
    )(ent_emb, h1, u2a, u2b, den, W1, b1[None, :], W2, b2[None, :],
      W3, b3[None, :])


def kernel(ent_emb, ent_transfer, rel_emb, rel_transfer, edge_index, edge_rel,
           edge_dir, W1, b1, W2, b2, W3, b3):
    head = edge_index[0].astype(i32)
    tail = edge_index[1].astype(i32)
    rel2 = edge_rel.astype(i32).reshape(E, 1)
    dir2 = edge_dir.reshape(E, 1)
    s16, ren = _prep(ent_emb, ent_transfer, rel_emb)

    halves = []
    for h in range(2):
        off = h * _E2
        gh, gt = _gather(ent_emb, head, tail, off)
        sh, st = _gather_s(s16, head, tail, off)
        halves.append((off, gh, gt, sh, st))

    scored = []
    for off, gh, gt, sh, st in halves:
        m1, ex16 = _score(gh, gt, sh, st, rel2[off:off + _E2],
                          dir2[off:off + _E2], rel_transfer, ren)
        scored.append((off, m1, ex16))

    u1s, dens = [], []
    for off, m1, ex16 in scored:
        u1s.append(_scatter_rows(m1, head, off))
        dens.append(_scatter_den(ex16, head, off))

    den = (dens[0][0, :N, 0:1] + dens[0][1, :N, 0:1]
           + dens[1][0, :N, 0:1] + dens[1][1, :N, 0:1])
    h1 = _combine(u1s[0].reshape(_NP, D)[:N], u1s[1].reshape(_NP, D)[:N], den)

    u2s = []
    for off, m1, ex16 in scored:
        g2 = _gather2(h1, tail, off)
        m2 = _mul(g2, ex16)
        u2s.append(_scatter_rows(m2, head, off))

    return _final(ent_emb, h1, u2s[0].reshape(_NP, D)[:N],
                  u2s[1].reshape(_NP, D)[:N], den,
                  W1, b1, W2, b2, W3, b3)
